# Initial kernel scaffold; baseline (speedup 1.0000x reference)
import jax
import jax.numpy as jnp
from jax.experimental import pallas as pl


def kernel(inputs, embbedL, embbedP):
    raise NotImplementedError("write your pallas kernel here")


# speedup vs baseline: 9.9622x; 9.9622x over previous
import functools

import jax
import jax.numpy as jnp
from jax import lax
from jax.experimental import pallas as pl
from jax.experimental.pallas import tpu as pltpu
from jax.experimental.pallas import tpu_sc as plsc

NW = 32
HALF = 100


def _emb_kernel(B, T, D, n_seq):
  mesh = plsc.VectorSubcoreMesh(
      core_axis_name="c", subcore_axis_name="s", num_cores=2, num_subcores=16)

  @functools.partial(
      pl.kernel,
      mesh=mesh,
      out_type=jax.ShapeDtypeStruct((B, T, D), jnp.float32),
      scratch_types=[
          pltpu.VMEM((n_seq, 2, HALF), jnp.int32),
          pltpu.VMEM_SHARED((T, D), jnp.float32),
          pltpu.VMEM((T, D), jnp.float32),
          pltpu.SemaphoreType.DMA,
      ],
  )
  def k(idx_hbm, tab_hbm, pos_hbm, out_hbm, idx_v, pos_sh, acc, sem):
    sid = lax.axis_index("s")
    wid = lax.axis_index("c") * 16 + sid
    base = wid * n_seq

    pltpu.sync_copy(idx_hbm.at[wid], idx_v)
    @pl.when(sid == 0)
    def _():
      pltpu.sync_copy(pos_hbm, pos_sh)
    plsc.subcore_barrier()

    def body(g, carry):
      pltpu.sync_copy(pos_sh, acc)
      cp0 = pltpu.async_copy(
          tab_hbm.at[idx_v.at[g, 0]], acc.at[pl.ds(0, HALF)], sem, add=True)
      cp1 = pltpu.async_copy(
          tab_hbm.at[idx_v.at[g, 1]], acc.at[pl.ds(HALF, HALF)], sem, add=True)
      cp0.wait()
      cp1.wait()
      pltpu.sync_copy(acc, out_hbm.at[base + g])
      return carry

    lax.fori_loop(0, n_seq, body, 0)

  return k


def kernel(inputs, embbedL, embbedP):
  B, T = inputs.shape
  V, D = embbedL.shape
  assert B % NW == 0 and T == 2 * HALF
  n_seq = B // NW

  idx = inputs.reshape(NW, n_seq, 2, HALF).astype(jnp.int32)
  return _emb_kernel(B, T, D, n_seq)(idx, embbedL, embbedP)


# speedup vs baseline: 14.1447x; 1.4198x over previous
import functools

import jax
import jax.numpy as jnp
from jax import lax
from jax.experimental import pallas as pl
from jax.experimental.pallas import tpu as pltpu
from jax.experimental.pallas import tpu_sc as plsc

NW = 32
HALF = 100
NBUF = 4


def _emb_kernel(B, T, D, n_seq):
  mesh = plsc.VectorSubcoreMesh(
      core_axis_name="c", subcore_axis_name="s", num_cores=2, num_subcores=16)

  @functools.partial(
      pl.kernel,
      mesh=mesh,
      out_type=jax.ShapeDtypeStruct((B, T, D), jnp.float32),
      scratch_types=[
          pltpu.VMEM((n_seq, 2, HALF), jnp.int32),
          pltpu.VMEM_SHARED((T, D), jnp.float32),
          [pltpu.VMEM((T, D), jnp.float32)] * NBUF,
          pltpu.SemaphoreType.DMA((NBUF,)),
          pltpu.SemaphoreType.DMA((NBUF,)),
          pltpu.SemaphoreType.DMA((NBUF,)),
      ],
  )
  def k(idx_hbm, tab_hbm, pos_hbm, out_hbm,
        idx_v, pos_sh, accs, ssem, gsem, osem):
    sid = lax.axis_index("s")
    wid = lax.axis_index("c") * 16 + sid
    base = wid * n_seq

    pltpu.sync_copy(idx_hbm.at[wid], idx_v)
    @pl.when(sid == 0)
    def _():
      pltpu.sync_copy(pos_hbm, pos_sh)
    plsc.subcore_barrier()

    def seed(b):
      pltpu.async_copy(pos_sh, accs[b], ssem.at[b])

    seed(0)
    seed(1)

    def round_body(r, carry):
      for b in range(NBUF):
        c = NBUF * r + b
        acc = accs[b]
        pltpu.make_async_copy(pos_sh, acc, ssem.at[b]).wait()
        cp0 = pltpu.async_copy(
            tab_hbm.at[idx_v.at[c, 0]], acc.at[pl.ds(0, HALF)],
            gsem.at[b], add=True)
        cp1 = pltpu.async_copy(
            tab_hbm.at[idx_v.at[c, 1]], acc.at[pl.ds(HALF, HALF)],
            gsem.at[b], add=True)
        cp0.wait()
        cp1.wait()
        pltpu.async_copy(acc, out_hbm.at[base + c], osem.at[b])
        b2 = (b + 2) % NBUF
        if b < 2:
          @pl.when(r > 0)
          def _():
            pltpu.make_async_copy(accs[b2], out_hbm.at[base], osem.at[b2]).wait()
          seed(b2)
        else:
          pltpu.make_async_copy(accs[b2], out_hbm.at[base], osem.at[b2]).wait()

          @pl.when(c + 2 < n_seq)
          def _():
            seed(b2)
      return carry

    lax.fori_loop(0, n_seq // NBUF, round_body, 0)

    for b in (NBUF - 2, NBUF - 1):
      pltpu.make_async_copy(accs[b], out_hbm.at[base], osem.at[b]).wait()

  return k


def kernel(inputs, embbedL, embbedP):
  B, T = inputs.shape
  V, D = embbedL.shape
  assert B % NW == 0 and T == 2 * HALF
  n_seq = B // NW
  assert n_seq % NBUF == 0

  idx = inputs.reshape(NW, n_seq, 2, HALF).astype(jnp.int32)
  return _emb_kernel(B, T, D, n_seq)(idx, embbedL, embbedP)


# speedup vs baseline: 15.0562x; 1.0644x over previous
import functools

import jax
import jax.numpy as jnp
from jax import lax
from jax.experimental import pallas as pl
from jax.experimental.pallas import tpu as pltpu
from jax.experimental.pallas import tpu_sc as plsc

NW = 32
HALF = 100
NBUF = 4


def _emb_kernel(B, T, D, n_seq):
  mesh = plsc.VectorSubcoreMesh(
      core_axis_name="c", subcore_axis_name="s", num_cores=2, num_subcores=16)
  n_rounds = n_seq // NBUF

  @functools.partial(
      pl.kernel,
      mesh=mesh,
      out_type=jax.ShapeDtypeStruct((B, T, D), jnp.float32),
      scratch_types=[
          pltpu.VMEM((n_seq, 2, HALF), jnp.int32),
          pltpu.VMEM_SHARED((T, D), jnp.float32),
          [pltpu.VMEM((T, D), jnp.float32)] * NBUF,
          pltpu.SemaphoreType.DMA((NBUF,)),
          pltpu.SemaphoreType.DMA((NBUF,)),
          pltpu.SemaphoreType.DMA((NBUF,)),
      ],
  )
  def k(idx_hbm, tab_hbm, pos_hbm, out_hbm,
        idx_v, pos_sh, accs, ssem, gsem, osem):
    sid = lax.axis_index("s")
    wid = lax.axis_index("c") * 16 + sid
    base = wid * n_seq

    pltpu.sync_copy(idx_hbm.at[wid], idx_v)
    @pl.when(sid == 0)
    def _():
      pltpu.sync_copy(pos_hbm, pos_sh)
    plsc.subcore_barrier()

    def seed(b):
      pltpu.async_copy(pos_sh, accs[b], ssem.at[b])

    def seed_wait(b):
      pltpu.make_async_copy(pos_sh, accs[b], ssem.at[b]).wait()

    def gather_start(c, b):
      pltpu.async_copy(tab_hbm.at[idx_v.at[c, 0]],
                       accs[b].at[pl.ds(0, HALF)], gsem.at[b], add=True)
      pltpu.async_copy(tab_hbm.at[idx_v.at[c, 1]],
                       accs[b].at[pl.ds(HALF, HALF)], gsem.at[b], add=True)

    def gather_wait(c, b):
      for h in range(2):
        pltpu.make_async_copy(tab_hbm.at[idx_v.at[c, h]],
                              accs[b].at[pl.ds(h * HALF, HALF)],
                              gsem.at[b]).wait()

    def out_wait(b):
      pltpu.make_async_copy(accs[b], out_hbm.at[base], osem.at[b]).wait()

    seed(0)
    seed(1)
    seed(2)
    seed_wait(0)
    gather_start(0, 0)

    def round_body(r, carry):
      for b in range(NBUF):
        c = NBUF * r + b
        b1, b2 = (b + 1) % NBUF, (b + 2) % NBUF
        if b == NBUF - 1:
          @pl.when(r < n_rounds - 1)
          def _():
            seed_wait(b1)
            gather_start(c + 1, b1)
        else:
          seed_wait(b1)
          gather_start(c + 1, b1)
        gather_wait(c, b)
        pltpu.async_copy(accs[b], out_hbm.at[base + c], osem.at[b])
        if b < 2:
          @pl.when(r > 0)
          def _():
            out_wait(b2)
          if b == 1:
            seed(b2)
          else:
            @pl.when(r > 0)
            def _():
              seed(b2)
        else:
          out_wait(b2)

          @pl.when(c + 2 < n_seq)
          def _():
            seed(b2)
      return carry

    lax.fori_loop(0, n_rounds, round_body, 0)

    for b in (NBUF - 2, NBUF - 1):
      out_wait(b)

  return k


def kernel(inputs, embbedL, embbedP):
  B, T = inputs.shape
  V, D = embbedL.shape
  assert B % NW == 0 and T == 2 * HALF
  n_seq = B // NW
  assert n_seq % NBUF == 0 and n_seq // NBUF >= 2

  idx = inputs.reshape(NW, n_seq, 2, HALF).astype(jnp.int32)
  return _emb_kernel(B, T, D, n_seq)(idx, embbedL, embbedP)
